# baseline (speedup 1.0000x reference)
import functools
import numpy as np
import jax
import jax.numpy as jnp
from jax import lax
from jax.experimental import pallas as pl
from jax.experimental.pallas import tpu as pltpu
from jax.experimental.pallas import tpu_sc as plsc

_NUM_STICK = 117
_NUM_TRIGGER = 99
_NUM_BUTTONS = 128
_WIDTH = 4 * _NUM_STICK + _NUM_TRIGGER + _NUM_BUTTONS
_OFFSETS = (0, _NUM_STICK, 2 * _NUM_STICK, 3 * _NUM_STICK,
            4 * _NUM_STICK, 4 * _NUM_STICK + _NUM_TRIGGER)

_B = 4096
_T = 20
_NCOMP = 6
_NC = 2
_NS = 16
_NW = _NC * _NS
_BPW = _B // _NW
_G = 4
_NITER = _BPW // _G
_NCODE = _G * _T * _NCOMP
_NCHUNKS = _NCODE // 16

def _scatter_block(blocks, xbuf, b, value):
    vals = jnp.full((16,), value, jnp.float32)
    bsp = jnp.full((16,), b, jnp.int32)
    lane = lax.iota(jnp.int32, 16)
    for c in range(_NCHUNKS):
        gci = lane + (16 * c)
        row = gci // _NCOMP
        gl = row // _T
        tl = row % _T
        comp = gci % _NCOMP
        offs = jnp.where(comp < 4, _NUM_STICK * comp,
                         4 * _NUM_STICK + _NUM_TRIGGER * (comp - 4))
        codes = plsc.load_gather(xbuf, [bsp, gl, tl, comp])
        col = codes + offs
        plsc.store_scatter(blocks, [bsp, gl, tl, col], vals)


def _sc_kernel(x_hbm, o_hbm, blocks, xbuf, sem0, sem1):
    sems = (sem0, sem1)
    wid = lax.axis_index("s") * _NC + lax.axis_index("c")
    wbase = wid * _BPW

    def copy_for(b, i):
        return pltpu.make_async_copy(
            blocks.at[b],
            o_hbm.at[pl.ds(wbase + i * _G, _G)],
            sems[b])

    zeros16 = jnp.zeros((16,), jnp.float32)
    tail_iota = lax.iota(jnp.int32, 16)
    tail_idx = tail_iota + (_WIDTH - _WIDTH % 16)
    tail_mask = tail_iota < (_WIDTH % 16)

    def zbody(r, carry):
        b = r // (_G * _T)
        g = (r // _T) % _G
        t = r % _T
        for k in range(_WIDTH // 16):
            blocks[b, g, t, pl.ds(16 * k, 16)] = zeros16
        plsc.store_scatter(
            blocks,
            [jnp.full((16,), b, jnp.int32), jnp.full((16,), g, jnp.int32),
             jnp.full((16,), t, jnp.int32), tail_idx],
            zeros16, mask=tail_mask)
        return carry

    lax.fori_loop(0, 2 * _G * _T, zbody, 0)

    for b in range(2):
        pltpu.make_async_copy(
            x_hbm.at[pl.ds(wbase + b * _G, _G)], xbuf.at[b], sems[b]).start()
        pltpu.make_async_copy(
            x_hbm.at[pl.ds(wbase + b * _G, _G)], xbuf.at[b], sems[b]).wait()
        _scatter_block(blocks, xbuf, b, 1.0)
        copy_for(b, b).start()

    def body(j, carry):
        for b in range(2):
            i = 2 * j + b
            copy_for(b, i - 2).wait()
            _scatter_block(blocks, xbuf, b, 0.0)
            pltpu.make_async_copy(
                x_hbm.at[pl.ds(wbase + i * _G, _G)], xbuf.at[b],
                sems[b]).start()
            pltpu.make_async_copy(
                x_hbm.at[pl.ds(wbase + i * _G, _G)], xbuf.at[b],
                sems[b]).wait()
            _scatter_block(blocks, xbuf, b, 1.0)
            copy_for(b, i).start()
        return carry

    lax.fori_loop(1, _NITER // 2, body, 0)

    for b in range(2):
        copy_for(b, _NITER - 2 + b).wait()


def kernel(x):
    mesh = plsc.VectorSubcoreMesh(core_axis_name="c", subcore_axis_name="s")
    f = functools.partial(
        pl.kernel,
        mesh=mesh,
        out_type=jax.ShapeDtypeStruct((_B, _T, _WIDTH), jnp.float32),
        scratch_types=[
            pltpu.VMEM((2, _G, _T, _WIDTH), jnp.float32),
            pltpu.VMEM((2, _G, _T, _NCOMP), jnp.int32),
            pltpu.SemaphoreType.DMA,
            pltpu.SemaphoreType.DMA,
        ],
        compiler_params=pltpu.CompilerParams(use_tc_tiling_on_sc=False, needs_layout_passes=False),
    )(_sc_kernel)
    return f(x.astype(jnp.int32))


# speedup vs baseline: 1.6757x; 1.6757x over previous
import functools
import jax
import jax.numpy as jnp
from jax import lax
from jax.experimental import pallas as pl
from jax.experimental.pallas import tpu as pltpu
from jax.experimental.pallas import tpu_sc as plsc

_NUM_STICK = 117
_NUM_TRIGGER = 99
_NUM_BUTTONS = 128
_WIDTH = 4 * _NUM_STICK + _NUM_TRIGGER + _NUM_BUTTONS

_B = 4096
_T = 20
_NCOMP = 6
_ST = 3
_LT = 6
_NC = 2
_NS = 16
_NW = _NC * _NS
_BPW = _B // _NW
_G = 2
_NITER = _BPW // _G
_NCHUNKS = _G * _T * _NCOMP // 16


def _scatter_block(blocks, xbuf, b, value):
    vals = jnp.full((16,), value, jnp.float32)
    bsp = jnp.full((16,), b, jnp.int32)
    lane = lax.iota(jnp.int32, 16)
    for c in range(_NCHUNKS):
        gci = lane + (16 * c)
        row = gci // _NCOMP
        gl = row // _T
        tl = row % _T
        comp = gci % _NCOMP
        offs = jnp.where(comp < 4, _NUM_STICK * comp,
                         4 * _NUM_STICK + _NUM_TRIGGER * (comp - 4))
        codes = plsc.load_gather(xbuf, [bsp, gl, tl, comp])
        col = codes + offs
        plsc.store_scatter(
            blocks,
            [bsp, gl, tl // 8, col // 128, tl % 8, col % 128],
            vals)


def _sc_kernel(x_hbm, o_hbm, blocks, xbuf, sem0, sem1):
    sems = (sem0, sem1)
    wid = lax.axis_index("s") * _NC + lax.axis_index("c")
    wbase = wid * _BPW

    def copy_for(b, i):
        return pltpu.make_async_copy(
            blocks.at[b],
            o_hbm.at[pl.ds(wbase + i * _G, _G)],
            sems[b])

    zeros16 = jnp.zeros((16,), jnp.float32)

    def zbody(q, carry):
        b = q // (_G * _ST * _LT)
        g = (q // (_ST * _LT)) % _G
        st = (q // _LT) % _ST
        lt = q % _LT
        for s in range(8):
            for m in range(8):
                blocks[b, g, st, lt, s, pl.ds(16 * m, 16)] = zeros16
        return carry

    lax.fori_loop(0, 2 * _G * _ST * _LT, zbody, 0)

    for b in range(2):
        pltpu.make_async_copy(
            x_hbm.at[pl.ds(wbase + b * _G, _G)], xbuf.at[b], sems[b]).start()
        pltpu.make_async_copy(
            x_hbm.at[pl.ds(wbase + b * _G, _G)], xbuf.at[b], sems[b]).wait()
        _scatter_block(blocks, xbuf, b, 1.0)
        copy_for(b, b).start()

    def body(j, carry):
        for b in range(2):
            i = 2 * j + b
            copy_for(b, i - 2).wait()
            _scatter_block(blocks, xbuf, b, 0.0)
            pltpu.make_async_copy(
                x_hbm.at[pl.ds(wbase + i * _G, _G)], xbuf.at[b],
                sems[b]).start()
            pltpu.make_async_copy(
                x_hbm.at[pl.ds(wbase + i * _G, _G)], xbuf.at[b],
                sems[b]).wait()
            _scatter_block(blocks, xbuf, b, 1.0)
            copy_for(b, i).start()
        return carry

    lax.fori_loop(1, _NITER // 2, body, 0)

    for b in range(2):
        copy_for(b, _NITER - 2 + b).wait()


def kernel(x):
    mesh = plsc.VectorSubcoreMesh(core_axis_name="c", subcore_axis_name="s")
    f = functools.partial(
        pl.kernel,
        mesh=mesh,
        out_type=jax.ShapeDtypeStruct((_B, _ST, _LT, 8, 128), jnp.float32),
        scratch_types=[
            pltpu.VMEM((2, _G, _ST, _LT, 8, 128), jnp.float32),
            pltpu.VMEM((2, _G, _T, _NCOMP), jnp.int32),
            pltpu.SemaphoreType.DMA,
            pltpu.SemaphoreType.DMA,
        ],
        compiler_params=pltpu.CompilerParams(
            use_tc_tiling_on_sc=False, needs_layout_passes=False),
    )(_sc_kernel)
    raw = f(x.astype(jnp.int32))
    out = raw.transpose(0, 1, 3, 2, 4).reshape(_B, 8 * _ST, 128 * _LT)
    return out[:, :_T, :_WIDTH]
